# baseline (device time: 56932 ns/iter reference)
import jax
import jax.numpy as jnp
from jax import lax
from jax.experimental import pallas as pl
from jax.experimental.pallas import tpu as pltpu

N_DEV = 4


def kernel(A, B):
    m, k = A.shape
    _, n = B.shape
    m_out = m // N_DEV

    def body(a_ref, b_ref, out_ref, acc_ref, comm_ref, send_sems, recv_sems):
        my = lax.axis_index("i")
        left = (my - 1) % N_DEV
        right = (my + 1) % N_DEV

        barrier_sem = pltpu.get_barrier_semaphore()
        for nbr in (left, right):
            pl.semaphore_signal(
                barrier_sem, inc=1,
                device_id=(nbr,), device_id_type=pl.DeviceIdType.MESH,
            )
        pl.semaphore_wait(barrier_sem, 2)

        b_bf = b_ref[...].astype(jnp.bfloat16)
        for j in range(N_DEV):
            c = (my - 1 - j) % N_DEV
            a_chunk = a_ref[pl.ds(c * m_out, m_out), :].astype(jnp.bfloat16)
            acc_ref[j] = jnp.dot(
                a_chunk, b_bf, preferred_element_type=jnp.float32
            )

        comm_ref[0] = acc_ref[0].astype(jnp.bfloat16)

        for s in range(N_DEV - 1):
            rdma = pltpu.make_async_remote_copy(
                src_ref=comm_ref.at[s],
                dst_ref=comm_ref.at[s + 1],
                send_sem=send_sems.at[s],
                recv_sem=recv_sems.at[s],
                device_id=(right,),
                device_id_type=pl.DeviceIdType.MESH,
            )
            rdma.start()
            rdma.wait()
            if s < N_DEV - 2:
                comm_ref[s + 1] = (
                    comm_ref[s + 1].astype(jnp.float32) + acc_ref[s + 1]
                ).astype(jnp.bfloat16)

        out_ref[...] = (
            comm_ref[N_DEV - 1].astype(jnp.float32) + acc_ref[N_DEV - 1]
        )

    return pl.pallas_call(
        body,
        out_shape=jax.ShapeDtypeStruct((m_out, n), jnp.float32),
        in_specs=[
            pl.BlockSpec(memory_space=pltpu.VMEM),
            pl.BlockSpec(memory_space=pltpu.VMEM),
        ],
        out_specs=pl.BlockSpec(memory_space=pltpu.VMEM),
        scratch_shapes=[
            pltpu.VMEM((N_DEV, m_out, n), jnp.float32),
            pltpu.VMEM((N_DEV, m_out, n), jnp.bfloat16),
            pltpu.SemaphoreType.DMA((N_DEV - 1,)),
            pltpu.SemaphoreType.DMA((N_DEV - 1,)),
        ],
        compiler_params=pltpu.CompilerParams(collective_id=0),
    )(A, B)


# device time: 39190 ns/iter; 1.4527x vs baseline; 1.4527x over previous
import jax
import jax.numpy as jnp
from jax import lax
from jax.experimental import pallas as pl
from jax.experimental.pallas import tpu as pltpu

N_DEV = 4


def kernel(A, B):
    m, k = A.shape
    _, n = B.shape
    m_out = m // N_DEV

    def body(a_ref, b_ref, out_ref, send_ref, recv_ref, send_sems, recv_sems):
        my = lax.axis_index("i")

        barrier_sem = pltpu.get_barrier_semaphore()
        for t in range(1, N_DEV):
            pl.semaphore_signal(
                barrier_sem, inc=1,
                device_id=((my + t) % N_DEV,),
                device_id_type=pl.DeviceIdType.MESH,
            )
        pl.semaphore_wait(barrier_sem, N_DEV - 1)

        b_bf = b_ref[...].astype(jnp.bfloat16)

        rdmas = {}
        for t in (2, 1, 3):
            c = (my + t) % N_DEV
            a_chunk = a_ref[pl.ds(c * m_out, m_out), :].astype(jnp.bfloat16)
            send_ref[t - 1] = jnp.dot(
                a_chunk, b_bf, preferred_element_type=jnp.float32
            ).astype(jnp.bfloat16)
            rdma = pltpu.make_async_remote_copy(
                src_ref=send_ref.at[t - 1],
                dst_ref=recv_ref.at[t - 1],
                send_sem=send_sems.at[t - 1],
                recv_sem=recv_sems.at[t - 1],
                device_id=(c,),
                device_id_type=pl.DeviceIdType.MESH,
            )
            rdma.start()
            rdmas[t] = rdma

        a_own = a_ref[pl.ds(my * m_out, m_out), :].astype(jnp.bfloat16)
        acc = jnp.dot(a_own, b_bf, preferred_element_type=jnp.float32)

        for t in (1, 3, 2):
            rdmas[t].wait_recv()
            acc = acc + recv_ref[t - 1].astype(jnp.float32)
        out_ref[...] = acc

        for t in (1, 2, 3):
            rdmas[t].wait_send()

    return pl.pallas_call(
        body,
        out_shape=jax.ShapeDtypeStruct((m_out, n), jnp.float32),
        in_specs=[
            pl.BlockSpec(memory_space=pltpu.VMEM),
            pl.BlockSpec(memory_space=pltpu.VMEM),
        ],
        out_specs=pl.BlockSpec(memory_space=pltpu.VMEM),
        scratch_shapes=[
            pltpu.VMEM((N_DEV - 1, m_out, n), jnp.bfloat16),
            pltpu.VMEM((N_DEV - 1, m_out, n), jnp.bfloat16),
            pltpu.SemaphoreType.DMA((N_DEV - 1,)),
            pltpu.SemaphoreType.DMA((N_DEV - 1,)),
        ],
        compiler_params=pltpu.CompilerParams(collective_id=0),
    )(A, B)


# device time: 35050 ns/iter; 1.6243x vs baseline; 1.1181x over previous
import jax
import jax.numpy as jnp
from jax import lax
from jax.experimental import pallas as pl
from jax.experimental.pallas import tpu as pltpu

N_DEV = 4


def kernel(A, B):
    m, k = A.shape
    _, n = B.shape
    m_out = m // N_DEV
    nh = n // 2

    def body(a_ref, b_ref, out_ref,
             sendA1, sendB1, sendA2, sendB2,
             recvA1, recvB1, recvA2, recvB2,
             send_sems, recv_sems):
        d = lax.axis_index("i")
        pA = d + 1 - 2 * (d % 2)
        pB = 3 - d
        c_last = 3 - pA

        barrier_sem = pltpu.get_barrier_semaphore()
        for nbr in (pA, pB):
            pl.semaphore_signal(
                barrier_sem, inc=1,
                device_id=(nbr,), device_id_type=pl.DeviceIdType.MESH,
            )
        pl.semaphore_wait(barrier_sem, 2)

        b_bf = b_ref[...].astype(jnp.bfloat16)
        bL = b_bf[:, :nh]
        bR = b_bf[:, nh:]

        def chunk(c):
            return a_ref[pl.ds(c * m_out, m_out), :].astype(jnp.bfloat16)

        def half_mm(c, b_half, out_dtype):
            r = jnp.dot(chunk(c), b_half, preferred_element_type=jnp.float32)
            return r.astype(out_dtype)

        def rdma(src, dst, slot, target):
            return pltpu.make_async_remote_copy(
                src_ref=src, dst_ref=dst,
                send_sem=send_sems.at[slot], recv_sem=recv_sems.at[slot],
                device_id=(target,), device_id_type=pl.DeviceIdType.MESH,
            )

        sendA1[0] = half_mm(pA, bL, jnp.bfloat16)
        sendA1[1] = half_mm(c_last, bL, jnp.bfloat16)
        rA1 = rdma(sendA1, recvA1, 0, pA)
        rA1.start()
        sendB1[0] = half_mm(pB, bR, jnp.bfloat16)
        sendB1[1] = half_mm(c_last, bR, jnp.bfloat16)
        rB1 = rdma(sendB1, recvB1, 1, pB)
        rB1.start()

        keepA0 = half_mm(d, bL, jnp.float32)
        keepA1 = half_mm(pB, bL, jnp.float32)
        keepB0 = half_mm(d, bR, jnp.float32)
        keepB1 = half_mm(pA, bR, jnp.float32)

        rA1.wait_recv()
        accA0 = keepA0 + recvA1[0].astype(jnp.float32)
        accA1 = keepA1 + recvA1[1].astype(jnp.float32)
        sendA2[...] = accA1.astype(jnp.bfloat16)
        rA2 = rdma(sendA2, recvA2, 2, pB)
        rA2.start()

        rB1.wait_recv()
        accB0 = keepB0 + recvB1[0].astype(jnp.float32)
        accB1 = keepB1 + recvB1[1].astype(jnp.float32)
        sendB2[...] = accB1.astype(jnp.bfloat16)
        rB2 = rdma(sendB2, recvB2, 3, pA)
        rB2.start()

        rA2.wait_recv()
        out_ref[:, :nh] = accA0 + recvA2[...].astype(jnp.float32)
        rB2.wait_recv()
        out_ref[:, nh:] = accB0 + recvB2[...].astype(jnp.float32)

        for r in (rA1, rB1, rA2, rB2):
            r.wait_send()

    return pl.pallas_call(
        body,
        out_shape=jax.ShapeDtypeStruct((m_out, n), jnp.float32),
        in_specs=[
            pl.BlockSpec(memory_space=pltpu.VMEM),
            pl.BlockSpec(memory_space=pltpu.VMEM),
        ],
        out_specs=pl.BlockSpec(memory_space=pltpu.VMEM),
        scratch_shapes=[
            pltpu.VMEM((2, m_out, nh), jnp.bfloat16),
            pltpu.VMEM((2, m_out, nh), jnp.bfloat16),
            pltpu.VMEM((m_out, nh), jnp.bfloat16),
            pltpu.VMEM((m_out, nh), jnp.bfloat16),
            pltpu.VMEM((2, m_out, nh), jnp.bfloat16),
            pltpu.VMEM((2, m_out, nh), jnp.bfloat16),
            pltpu.VMEM((m_out, nh), jnp.bfloat16),
            pltpu.VMEM((m_out, nh), jnp.bfloat16),
            pltpu.SemaphoreType.DMA((4,)),
            pltpu.SemaphoreType.DMA((4,)),
        ],
        compiler_params=pltpu.CompilerParams(collective_id=0),
    )(A, B)


# device time: 34119 ns/iter; 1.6686x vs baseline; 1.0273x over previous
import jax
import jax.numpy as jnp
from jax import lax
from jax.experimental import pallas as pl
from jax.experimental.pallas import tpu as pltpu

N_DEV = 4


def kernel(A, B):
    m, k = A.shape
    _, n = B.shape
    m_out = m // N_DEV
    nh = n // 2

    def body(a_ref, b_ref, out_ref,
             sendA1, sendB1, sendA2, sendB2,
             recvA1, recvB1, recvA2, recvB2,
             send_sems, recv_sems):
        d = lax.axis_index("i")
        pA = d + 1 - 2 * (d % 2)
        pB = 3 - d
        c_last = 3 - pA

        barrier_sem = pltpu.get_barrier_semaphore()
        for nbr in (pA, pB):
            pl.semaphore_signal(
                barrier_sem, inc=1,
                device_id=(nbr,), device_id_type=pl.DeviceIdType.MESH,
            )
        pl.semaphore_wait(barrier_sem, 2)

        b_bf = b_ref[...].astype(jnp.bfloat16)
        bL = b_bf[:, :nh]
        bR = b_bf[:, nh:]

        def chunk(c):
            return a_ref[pl.ds(c * m_out, m_out), :].astype(jnp.bfloat16)

        def half_mm(c, b_half, out_dtype):
            r = jnp.dot(chunk(c), b_half, preferred_element_type=jnp.float32)
            return r.astype(out_dtype)

        def rdma(src, dst, slot, target):
            return pltpu.make_async_remote_copy(
                src_ref=src, dst_ref=dst,
                send_sem=send_sems.at[slot], recv_sem=recv_sems.at[slot],
                device_id=(target,), device_id_type=pl.DeviceIdType.MESH,
            )

        sendA1[0] = half_mm(pA, bL, jnp.bfloat16)
        rA1a = rdma(sendA1.at[0], recvA1.at[0], 0, pA)
        rA1a.start()
        sendB1[0] = half_mm(pB, bR, jnp.bfloat16)
        rB1a = rdma(sendB1.at[0], recvB1.at[0], 1, pB)
        rB1a.start()
        sendA1[1] = half_mm(c_last, bL, jnp.bfloat16)
        rA1b = rdma(sendA1.at[1], recvA1.at[1], 2, pA)
        rA1b.start()
        sendB1[1] = half_mm(c_last, bR, jnp.bfloat16)
        rB1b = rdma(sendB1.at[1], recvB1.at[1], 3, pB)
        rB1b.start()

        keepA1 = half_mm(pB, bL, jnp.float32)
        keepB1 = half_mm(pA, bR, jnp.float32)

        rA1b.wait_recv()
        sendA2[...] = (keepA1 + recvA1[1].astype(jnp.float32)).astype(
            jnp.bfloat16
        )
        rA2 = rdma(sendA2, recvA2, 4, pB)
        rA2.start()
        rB1b.wait_recv()
        sendB2[...] = (keepB1 + recvB1[1].astype(jnp.float32)).astype(
            jnp.bfloat16
        )
        rB2 = rdma(sendB2, recvB2, 5, pA)
        rB2.start()

        keepA0 = half_mm(d, bL, jnp.float32)
        keepB0 = half_mm(d, bR, jnp.float32)
        rA1a.wait_recv()
        accA0 = keepA0 + recvA1[0].astype(jnp.float32)
        rB1a.wait_recv()
        accB0 = keepB0 + recvB1[0].astype(jnp.float32)

        rA2.wait_recv()
        out_ref[:, :nh] = accA0 + recvA2[...].astype(jnp.float32)
        rB2.wait_recv()
        out_ref[:, nh:] = accB0 + recvB2[...].astype(jnp.float32)

        for r in (rA1a, rB1a, rA1b, rB1b, rA2, rB2):
            r.wait_send()

    return pl.pallas_call(
        body,
        out_shape=jax.ShapeDtypeStruct((m_out, n), jnp.float32),
        in_specs=[
            pl.BlockSpec(memory_space=pltpu.VMEM),
            pl.BlockSpec(memory_space=pltpu.VMEM),
        ],
        out_specs=pl.BlockSpec(memory_space=pltpu.VMEM),
        scratch_shapes=[
            pltpu.VMEM((2, m_out, nh), jnp.bfloat16),
            pltpu.VMEM((2, m_out, nh), jnp.bfloat16),
            pltpu.VMEM((m_out, nh), jnp.bfloat16),
            pltpu.VMEM((m_out, nh), jnp.bfloat16),
            pltpu.VMEM((2, m_out, nh), jnp.bfloat16),
            pltpu.VMEM((2, m_out, nh), jnp.bfloat16),
            pltpu.VMEM((m_out, nh), jnp.bfloat16),
            pltpu.VMEM((m_out, nh), jnp.bfloat16),
            pltpu.SemaphoreType.DMA((6,)),
            pltpu.SemaphoreType.DMA((6,)),
        ],
        compiler_params=pltpu.CompilerParams(collective_id=0),
    )(A, B)
